# baseline (speedup 1.0000x reference)
import jax
import jax.numpy as jnp
from jax import lax
from jax.experimental import pallas as pl
from jax.experimental.pallas import tpu as pltpu, tpu_sc as plsc

B, N, D = 32, 4096, 64
NC, NS, L = 2, 16, 16
NVEC = N // L
TD = D // 8
TN = N // 128
NBUF = 2
HALF = TN // 2


def _sc_body(x_hbm, w_hbm, ll_hbm, pi_hbm, inds_hbm,
             mean_hbm, xr_hbm, wr_hbm, llr_hbm, pir_hbm,
             inds_v, w_v, ll_v, pi_v, wr_v, llr_v, pir_v, acc_v,
             ibuf0, ibuf1, obuf0, obuf1,
             sem_in0, sem_in1, sem_out0, sem_out1, sem_s):
  wid = lax.axis_index("s") * NC + lax.axis_index("c")
  tb = wid // 8
  br = wid % 8
  xb = x_hbm.at[wid]
  xrb = xr_hbm.at[wid]
  ibufs = (ibuf0, ibuf1)
  obufs = (obuf0, obuf1)
  sems_in = (sem_in0, sem_in1)
  sems_out = (sem_out0, sem_out1)
  wexp_v = w_v

  in_dma = {}
  for g in range(NBUF):
    in_dma[g] = pltpu.async_copy(xb.at[g], ibufs[g], sems_in[g])

  pltpu.sync_copy(inds_hbm.at[tb, :, br], inds_v)
  pltpu.sync_copy(w_hbm.at[tb, :, br], w_v)
  pltpu.sync_copy(ll_hbm.at[tb, :, br], ll_v)
  pltpu.sync_copy(pi_hbm.at[tb, :, br], pi_v)

  def gather_body(j, mx):
    tn_j = j // 8
    nc_j = (j % 8) * L
    idx16 = inds_v[tn_j, pl.ds(nc_j, L)]
    hi = jax.lax.shift_right_logical(idx16, 7)
    lo = jax.lax.bitwise_and(idx16, 127)
    wr16 = plsc.load_gather(w_v, [hi, lo])
    wr_v[tn_j, pl.ds(nc_j, L)] = wr16
    llr_v[tn_j, pl.ds(nc_j, L)] = plsc.load_gather(ll_v, [hi, lo])
    pir_v[tn_j, pl.ds(nc_j, L)] = plsc.load_gather(pi_v, [hi, lo])
    return jnp.maximum(mx, wr16)

  mx16 = plsc.parallel_loop(
      0, NVEC, 1, unroll=2,
      carry=jnp.full((L,), -jnp.inf, jnp.float32))(gather_body)
  m = lax.reduce_max_p.bind(mx16, axes=(0,))

  def exp_body(j, s):
    tn_j = j // 8
    nc_j = (j % 8) * L
    e16 = jnp.exp(wr_v[tn_j, pl.ds(nc_j, L)] - m)
    wexp_v[tn_j, pl.ds(nc_j, L)] = e16
    return s + e16

  s16 = plsc.parallel_loop(
      0, NVEC, 1, unroll=2,
      carry=jnp.zeros((L,), jnp.float32))(exp_body)
  inv_z = (jnp.full((L,), 1.0, jnp.float32) /
           jnp.full((L,), lax.reduce_sum_p.bind(s16, axes=(0,)), jnp.float32))

  pltpu.sync_copy(wr_v, wr_hbm.at[tb, :, br])
  pltpu.sync_copy(llr_v, llr_hbm.at[tb, :, br])
  pltpu.sync_copy(pir_v, pir_hbm.at[tb, :, br])

  lane0 = lax.iota(jnp.int32, L) == 0

  def slab_pair(i, _):
    for iph in range(NBUF):
      g = i * NBUF + iph
      ibuf = ibufs[iph]
      pltpu.make_async_copy(xb.at[g], ibuf, sems_in[iph]).wait()

      accs = tuple(jnp.zeros((L,), jnp.float32) for _ in range(8))
      for h in range(2):
        obuf = obufs[h]

        @pl.when(g > 0)
        def _(obuf=obuf, h=h, g=g):
          pltpu.make_async_copy(
              obuf, xrb.at[g - 1, pl.ds(h * HALF, HALF)],
              sems_out[h]).wait()

        def half_body(j, accs, ibuf=ibuf, obuf=obuf, h=h):
          jj = h * (NVEC // 2) + j
          tn_j = jj // 8
          nc_j = (jj % 8) * L
          idx16 = inds_v[tn_j, pl.ds(nc_j, L)]
          we16 = wexp_v[tn_j, pl.ds(nc_j, L)]
          hi = jax.lax.shift_right_logical(idx16, 7)
          lo = jax.lax.bitwise_and(idx16, 127)
          tn_o = j // 8
          nc_o = (j % 8) * L
          new = []
          for r in range(8):
            g16 = plsc.load_gather(
                ibuf, [hi, jnp.full((L,), r, jnp.int32), lo])
            obuf[tn_o, r, pl.ds(nc_o, L)] = g16
            new.append(accs[r] + we16 * g16)
          return tuple(new)

        accs = plsc.parallel_loop(0, NVEC // 2, 1, unroll=3,
                                  carry=accs)(half_body)
        pltpu.async_copy(
            obuf, xrb.at[g, pl.ds(h * HALF, HALF)], sems_out[h])

      @pl.when(g + NBUF < TD)
      def _(g=g, ibuf=ibuf, iph=iph):
        pltpu.async_copy(xb.at[g + NBUF], ibuf, sems_in[iph])

      for r in range(8):
        s = lax.reduce_sum_p.bind(accs[r], axes=(0,))
        svec = jnp.full((L,), s, jnp.float32) * inv_z
        plsc.store_scatter(acc_v, [jnp.full((L,), g * 8 + r, jnp.int32)],
                           svec, mask=lane0)
    return 0

  lax.fori_loop(0, TD // NBUF, slab_pair, 0)
  for h in range(2):
    pltpu.make_async_copy(
        obufs[h], xrb.at[TD - 1, pl.ds(h * HALF, HALF)], sems_out[h]).wait()
  pltpu.sync_copy(acc_v, mean_hbm.at[wid])


@jax.jit
def kernel(x, w, ll, prev_inds, inds):
  inds32 = inds.astype(jnp.int32)
  pi32 = prev_inds.astype(jnp.int32)

  def view_x(a):
    return a.reshape(B, TN, 128, TD, 8).transpose(0, 3, 1, 4, 2)

  def view_s(a):
    return a.reshape(4, 8, TN, 128).transpose(0, 2, 1, 3)

  def unview_s(a):
    return a.transpose(0, 2, 1, 3).reshape(B, N)

  mesh = plsc.VectorSubcoreMesh(core_axis_name="c", subcore_axis_name="s")
  run = pl.kernel(
      _sc_body,
      out_type=(
          jax.ShapeDtypeStruct((B, D), jnp.float32),
          jax.ShapeDtypeStruct((B, TD, TN, 8, 128), jnp.float32),
          jax.ShapeDtypeStruct((4, TN, 8, 128), jnp.float32),
          jax.ShapeDtypeStruct((4, TN, 8, 128), jnp.float32),
          jax.ShapeDtypeStruct((4, TN, 8, 128), jnp.int32),
      ),
      mesh=mesh,
      compiler_params=pltpu.CompilerParams(needs_layout_passes=False,
                                           use_tc_tiling_on_sc=False),
      scratch_types=[
          pltpu.VMEM((TN, 128), jnp.int32),
          pltpu.VMEM((TN, 128), jnp.float32),
          pltpu.VMEM((TN, 128), jnp.float32),
          pltpu.VMEM((TN, 128), jnp.int32),
          pltpu.VMEM((TN, 128), jnp.float32),
          pltpu.VMEM((TN, 128), jnp.float32),
          pltpu.VMEM((TN, 128), jnp.int32),
          pltpu.VMEM((D,), jnp.float32),
          pltpu.VMEM((TN, 8, 128), jnp.float32),
          pltpu.VMEM((TN, 8, 128), jnp.float32),
          pltpu.VMEM((HALF, 8, 128), jnp.float32),
          pltpu.VMEM((HALF, 8, 128), jnp.float32),
          pltpu.SemaphoreType.DMA,
          pltpu.SemaphoreType.DMA,
          pltpu.SemaphoreType.DMA,
          pltpu.SemaphoreType.DMA,
          pltpu.SemaphoreType.DMA,
      ],
  )
  mean, xr5, wr4, llr4, pir4 = run(view_x(x), view_s(w), view_s(ll),
                                   view_s(pi32), view_s(inds32))
  x_r = xr5.transpose(0, 2, 4, 1, 3).reshape(B, N, D)
  return (mean, x_r, unview_s(wr4), unview_s(llr4),
          unview_s(pir4).astype(prev_inds.dtype))


# speedup vs baseline: 1.0957x; 1.0957x over previous
import jax
import jax.numpy as jnp
from jax import lax
from jax.experimental import pallas as pl
from jax.experimental.pallas import tpu as pltpu, tpu_sc as plsc

B, N, D = 32, 4096, 64
NC, NS, L = 2, 16, 16
NVEC = N // L
TD = D // 8
TN = N // 128
NBUF = 2
HALF = TN // 2


def _sc_body(x_hbm, w_hbm, ll_hbm, pi_hbm, inds_hbm,
             mean_hbm, xr_hbm, wr_hbm, llr_hbm, pir_hbm,
             inds_v, w_v, ll_v, pi_v, wr_v, llr_v, pir_v, acc_v,
             ibuf0, ibuf1, obuf0, obuf1,
             sem_in0, sem_in1, sem_out0, sem_out1, sem_s):
  wid = lax.axis_index("s") * NC + lax.axis_index("c")
  tb = wid // 8
  br = wid % 8
  xb = x_hbm.at[wid]
  xrb = xr_hbm.at[wid]
  ibufs = (ibuf0, ibuf1)
  obufs = (obuf0, obuf1)
  sems_in = (sem_in0, sem_in1)
  sems_out = (sem_out0, sem_out1)
  wexp_v = w_v

  in_dma = {}
  for g in range(NBUF):
    in_dma[g] = pltpu.async_copy(xb.at[g], ibufs[g], sems_in[g])

  stage = [pltpu.async_copy(inds_hbm.at[tb, :, br], inds_v, sem_s),
           pltpu.async_copy(w_hbm.at[tb, :, br], w_v, sem_s),
           pltpu.async_copy(ll_hbm.at[tb, :, br], ll_v, sem_s),
           pltpu.async_copy(pi_hbm.at[tb, :, br], pi_v, sem_s)]
  for cp in stage:
    cp.wait()

  def gather_body(j, mx):
    tn_j = j // 8
    nc_j = (j % 8) * L
    idx16 = inds_v[tn_j, pl.ds(nc_j, L)]
    hi = jax.lax.shift_right_logical(idx16, 7)
    lo = jax.lax.bitwise_and(idx16, 127)
    wr16 = plsc.load_gather(w_v, [hi, lo])
    wr_v[tn_j, pl.ds(nc_j, L)] = wr16
    llr_v[tn_j, pl.ds(nc_j, L)] = plsc.load_gather(ll_v, [hi, lo])
    pir_v[tn_j, pl.ds(nc_j, L)] = plsc.load_gather(pi_v, [hi, lo])
    return jnp.maximum(mx, wr16)

  mx16 = plsc.parallel_loop(
      0, NVEC, 1, unroll=2,
      carry=jnp.full((L,), -jnp.inf, jnp.float32))(gather_body)
  m = lax.reduce_max_p.bind(mx16, axes=(0,))

  def exp_body(j, s):
    tn_j = j // 8
    nc_j = (j % 8) * L
    e16 = jnp.exp(wr_v[tn_j, pl.ds(nc_j, L)] - m)
    wexp_v[tn_j, pl.ds(nc_j, L)] = e16
    return s + e16

  s16 = plsc.parallel_loop(
      0, NVEC, 1, unroll=2,
      carry=jnp.zeros((L,), jnp.float32))(exp_body)
  inv_z = (jnp.full((L,), 1.0, jnp.float32) /
           jnp.full((L,), lax.reduce_sum_p.bind(s16, axes=(0,)), jnp.float32))

  small_out = [pltpu.async_copy(wr_v, wr_hbm.at[tb, :, br], sem_s),
               pltpu.async_copy(llr_v, llr_hbm.at[tb, :, br], sem_s),
               pltpu.async_copy(pir_v, pir_hbm.at[tb, :, br], sem_s)]

  lane0 = lax.iota(jnp.int32, L) == 0

  def slab_pair(i, _):
    for iph in range(NBUF):
      g = i * NBUF + iph
      ibuf = ibufs[iph]
      pltpu.make_async_copy(xb.at[g], ibuf, sems_in[iph]).wait()

      accs = tuple(jnp.zeros((L,), jnp.float32) for _ in range(8))
      for h in range(2):
        obuf = obufs[h]

        @pl.when(g > 0)
        def _(obuf=obuf, h=h, g=g):
          pltpu.make_async_copy(
              obuf, xrb.at[g - 1, pl.ds(h * HALF, HALF)],
              sems_out[h]).wait()

        def half_body(j, accs, ibuf=ibuf, obuf=obuf, h=h):
          jj = h * (NVEC // 2) + j
          tn_j = jj // 8
          nc_j = (jj % 8) * L
          idx16 = inds_v[tn_j, pl.ds(nc_j, L)]
          we16 = wexp_v[tn_j, pl.ds(nc_j, L)]
          hi = jax.lax.shift_right_logical(idx16, 7)
          lo = jax.lax.bitwise_and(idx16, 127)
          tn_o = j // 8
          nc_o = (j % 8) * L
          new = []
          for r in range(8):
            g16 = plsc.load_gather(
                ibuf, [hi, jnp.full((L,), r, jnp.int32), lo])
            obuf[tn_o, r, pl.ds(nc_o, L)] = g16
            new.append(accs[r] + we16 * g16)
          return tuple(new)

        accs = plsc.parallel_loop(0, NVEC // 2, 1, unroll=2,
                                  carry=accs)(half_body)
        pltpu.async_copy(
            obuf, xrb.at[g, pl.ds(h * HALF, HALF)], sems_out[h])

      @pl.when(g + NBUF < TD)
      def _(g=g, ibuf=ibuf, iph=iph):
        pltpu.async_copy(xb.at[g + NBUF], ibuf, sems_in[iph])

      for r in range(8):
        s = lax.reduce_sum_p.bind(accs[r], axes=(0,))
        svec = jnp.full((L,), s, jnp.float32) * inv_z
        plsc.store_scatter(acc_v, [jnp.full((L,), g * 8 + r, jnp.int32)],
                           svec, mask=lane0)
    return 0

  lax.fori_loop(0, TD // NBUF, slab_pair, 0)
  for h in range(2):
    pltpu.make_async_copy(
        obufs[h], xrb.at[TD - 1, pl.ds(h * HALF, HALF)], sems_out[h]).wait()
  for cp in small_out:
    cp.wait()
  pltpu.sync_copy(acc_v, mean_hbm.at[wid])


@jax.jit
def kernel(x, w, ll, prev_inds, inds):
  inds32 = inds.astype(jnp.int32)
  pi32 = prev_inds.astype(jnp.int32)

  def view_x(a):
    return a.reshape(B, TN, 128, TD, 8).transpose(0, 3, 1, 4, 2)

  def view_s(a):
    return a.reshape(4, 8, TN, 128).transpose(0, 2, 1, 3)

  def unview_s(a):
    return a.transpose(0, 2, 1, 3).reshape(B, N)

  mesh = plsc.VectorSubcoreMesh(core_axis_name="c", subcore_axis_name="s")
  run = pl.kernel(
      _sc_body,
      out_type=(
          jax.ShapeDtypeStruct((B, D), jnp.float32),
          jax.ShapeDtypeStruct((B, TD, TN, 8, 128), jnp.float32),
          jax.ShapeDtypeStruct((4, TN, 8, 128), jnp.float32),
          jax.ShapeDtypeStruct((4, TN, 8, 128), jnp.float32),
          jax.ShapeDtypeStruct((4, TN, 8, 128), jnp.int32),
      ),
      mesh=mesh,
      compiler_params=pltpu.CompilerParams(needs_layout_passes=False,
                                           use_tc_tiling_on_sc=False),
      scratch_types=[
          pltpu.VMEM((TN, 128), jnp.int32),
          pltpu.VMEM((TN, 128), jnp.float32),
          pltpu.VMEM((TN, 128), jnp.float32),
          pltpu.VMEM((TN, 128), jnp.int32),
          pltpu.VMEM((TN, 128), jnp.float32),
          pltpu.VMEM((TN, 128), jnp.float32),
          pltpu.VMEM((TN, 128), jnp.int32),
          pltpu.VMEM((D,), jnp.float32),
          pltpu.VMEM((TN, 8, 128), jnp.float32),
          pltpu.VMEM((TN, 8, 128), jnp.float32),
          pltpu.VMEM((HALF, 8, 128), jnp.float32),
          pltpu.VMEM((HALF, 8, 128), jnp.float32),
          pltpu.SemaphoreType.DMA,
          pltpu.SemaphoreType.DMA,
          pltpu.SemaphoreType.DMA,
          pltpu.SemaphoreType.DMA,
          pltpu.SemaphoreType.DMA,
      ],
  )
  mean, xr5, wr4, llr4, pir4 = run(view_x(x), view_s(w), view_s(ll),
                                   view_s(pi32), view_s(inds32))
  x_r = xr5.transpose(0, 2, 4, 1, 3).reshape(B, N, D)
  return (mean, x_r, unview_s(wr4), unview_s(llr4),
          unview_s(pir4).astype(prev_inds.dtype))
